# Initial kernel scaffold; baseline (speedup 1.0000x reference)
import functools

import jax
import jax.numpy as jnp
from jax import lax
from jax.experimental import pallas as pl
from jax.experimental.pallas import tpu as pltpu
from jax.experimental.pallas import tpu_sc as plsc

N = 10000
E = 320000
D = 128
H = 16

NC, NS, L = 2, 16, 16
NW = NC * NS
CH = 128
CPW = (E + NW * CH - 1) // (NW * CH)
EPAD = NW * CPW * CH
NPAD = 10240
RPT = NPAD // NS
GRT = N // NS

_f32 = jnp.float32


def _sc_mesh():
    return plsc.VectorSubcoreMesh(
        core_axis_name="c", subcore_axis_name="s", num_cores=NC, num_subcores=NS
    )


def _zero_vmem(zero_v, rows):
    def body(i, carry):
        zero_v[i] = jnp.zeros((L,), _f32)
        return carry
    lax.fori_loop(0, rows, body, 0)


def _deg_body(dst_hbm, deg_hbm, deg_sh, dst_v, one_v, zero_v):
    cid = lax.axis_index("c")
    sid = lax.axis_index("s")
    wid = cid * NS + sid

    _zero_vmem(zero_v, RPT)
    pltpu.sync_copy(zero_v, deg_sh.at[pl.ds(sid * RPT, RPT)])

    def fill_ones(i, carry):
        one_v[i] = jnp.ones((L,), _f32)
        return carry
    lax.fori_loop(0, CH, fill_ones, 0)

    pltpu.sync_copy(dst_hbm.at[pl.ds(wid * CPW, CPW)], dst_v)
    plsc.subcore_barrier()

    def chunk(j, carry):
        pltpu.sync_copy(one_v, deg_sh.at[dst_v.at[j]], add=True)
        return carry
    lax.fori_loop(0, CPW, chunk, 0)

    plsc.subcore_barrier()
    pltpu.sync_copy(deg_sh.at[pl.ds(sid * RPT, RPT)],
                    deg_hbm.at[cid, pl.ds(sid * RPT, RPT)])


def _deg_call(dst2d):
    fn = pl.kernel(
        _deg_body,
        out_type=jax.ShapeDtypeStruct((NC, NPAD, H), _f32),
        mesh=_sc_mesh(),
        scratch_types=[
            pltpu.VMEM_SHARED((NPAD, H), _f32),
            pltpu.VMEM((CPW, CH), jnp.int32),
            pltpu.VMEM((CH, H), _f32),
            pltpu.VMEM((RPT, H), _f32),
        ],
    )
    return fn(dst2d)


def _msg_body(g_hbm, src_hbm, dst_hbm, acc_hbm,
              g_sh, acc_sh, src_v, dst_v, msg_v, zero_v):
    cid = lax.axis_index("c")
    sid = lax.axis_index("s")
    wid = cid * NS + sid

    _zero_vmem(zero_v, RPT)
    pltpu.sync_copy(zero_v, acc_sh.at[pl.ds(sid * RPT, RPT)])
    pltpu.sync_copy(g_hbm.at[pl.ds(sid * GRT, GRT)],
                    g_sh.at[pl.ds(sid * GRT, GRT)])
    pltpu.sync_copy(src_hbm.at[pl.ds(wid * CPW, CPW)], src_v)
    pltpu.sync_copy(dst_hbm.at[pl.ds(wid * CPW, CPW)], dst_v)
    plsc.subcore_barrier()

    def chunk(j, carry):
        pltpu.sync_copy(g_sh.at[src_v.at[j]], msg_v)
        pltpu.sync_copy(msg_v, acc_sh.at[dst_v.at[j]], add=True)
        return carry
    lax.fori_loop(0, CPW, chunk, 0)

    plsc.subcore_barrier()
    pltpu.sync_copy(acc_sh.at[pl.ds(sid * RPT, RPT)],
                    acc_hbm.at[cid, pl.ds(sid * RPT, RPT)])


def _msg_call(g, src2d, dst2d):
    fn = pl.kernel(
        _msg_body,
        out_type=jax.ShapeDtypeStruct((NC, NPAD, H), _f32),
        mesh=_sc_mesh(),
        scratch_types=[
            pltpu.VMEM_SHARED((N, H), _f32),
            pltpu.VMEM_SHARED((NPAD, H), _f32),
            pltpu.VMEM((CPW, CH), jnp.int32),
            pltpu.VMEM((CPW, CH), jnp.int32),
            pltpu.VMEM((CH, H), _f32),
            pltpu.VMEM((RPT, H), _f32),
        ],
    )
    return fn(g, src2d, dst2d)


def _k1_body(x_ref, w1_ref, deg_ref, g1_ref, dis_ref):
    dis = lax.rsqrt(deg_ref[0, :N, :] + deg_ref[1, :N, :] + 1.0)
    h = jnp.dot(x_ref[...], w1_ref[...], preferred_element_type=_f32)
    g1_ref[...] = h * dis
    dis_ref[...] = dis


def _k1(x, W1, deg):
    return pl.pallas_call(
        _k1_body,
        out_shape=(
            jax.ShapeDtypeStruct((N, H), _f32),
            jax.ShapeDtypeStruct((N, H), _f32),
        ),
    )(x, W1, deg)


def _k2_body(acc_ref, g1_ref, dis_ref, b1_ref, w2_ref, g2_ref):
    s = acc_ref[0, :N, :] + acc_ref[1, :N, :] + g1_ref[...]
    a1 = jnp.maximum(dis_ref[...] * s + b1_ref[...], 0.0)
    h2 = jnp.dot(a1, w2_ref[...], preferred_element_type=_f32)
    g2_ref[...] = h2 * dis_ref[...]


def _k2(acc1, g1, dis, b1, W2):
    return pl.pallas_call(
        _k2_body,
        out_shape=jax.ShapeDtypeStruct((N, H), _f32),
    )(acc1, g1, dis, b1, W2)


def _k3_body(acc_ref, g2_ref, dis_ref, b2_ref, out_ref):
    s = acc_ref[0, :N, :] + acc_ref[1, :N, :] + g2_ref[...]
    out_ref[...] = dis_ref[...] * s + b2_ref[...]


def _k3(acc2, g2, dis, b2):
    return pl.pallas_call(
        _k3_body,
        out_shape=jax.ShapeDtypeStruct((N, H), _f32),
    )(acc2, g2, dis, b2)


def kernel(x, edge_index, W1, b1, W2, b2):
    ei = edge_index.astype(jnp.int32)
    src, dst = ei[0], ei[1]
    npad_e = EPAD - E
    padi = jnp.arange(npad_e, dtype=jnp.int32)
    src_p = jnp.concatenate([src, padi % 16]).reshape(NW * CPW, CH)
    dst_p = jnp.concatenate([dst, N + (padi % 16)]).reshape(NW * CPW, CH)

    deg = _deg_call(dst_p)
    g1, dis = _k1(x, W1, deg)
    acc1 = _msg_call(g1, src_p, dst_p)
    g2 = _k2(acc1, g1, dis, b1.reshape(1, H), W2)
    acc2 = _msg_call(g2, src_p, dst_p)
    return _k3(acc2, g2, dis, b2.reshape(1, H))


# speedup vs baseline: 51.1613x; 51.1613x over previous
import functools

import jax
import jax.numpy as jnp
from jax import lax
from jax.experimental import pallas as pl
from jax.experimental.pallas import tpu as pltpu
from jax.experimental.pallas import tpu_sc as plsc

N = 10000
E = 320000
D = 128
H = 16

NC, NS, L = 2, 16, 16
NW = NC * NS
CH = 128
CPW = (E + NW * CH - 1) // (NW * CH)
EPAD = NW * CPW * CH
NPAD = 10240
RPT = NPAD // NS

_f32 = jnp.float32


def _sc_mesh():
    return plsc.VectorSubcoreMesh(
        core_axis_name="c", subcore_axis_name="s", num_cores=NC, num_subcores=NS
    )


def _zero_vmem(zero_v, rows):
    def body(i, carry):
        zero_v[i] = jnp.zeros((L,), _f32)
        return carry
    lax.fori_loop(0, rows, body, 0)


def _deg_body(dst_hbm, deg_hbm, deg_sh, dst_v, one_v, zero_v):
    cid = lax.axis_index("c")
    sid = lax.axis_index("s")
    wid = cid * NS + sid

    _zero_vmem(zero_v, RPT)
    pltpu.sync_copy(zero_v, deg_sh.at[pl.ds(sid * RPT, RPT)])

    def fill_ones(i, carry):
        one_v[i] = jnp.ones((L,), _f32)
        return carry
    lax.fori_loop(0, CH, fill_ones, 0)

    pltpu.sync_copy(dst_hbm.at[wid], dst_v)
    plsc.subcore_barrier()

    def chunk(j, carry):
        pltpu.sync_copy(one_v, deg_sh.at[dst_v.at[j]], add=True)
        return carry
    lax.fori_loop(0, CPW, chunk, 0)

    plsc.subcore_barrier()
    pltpu.sync_copy(deg_sh.at[pl.ds(sid * RPT, RPT)],
                    deg_hbm.at[cid, pl.ds(sid * RPT, RPT)])


def _deg_call(dst2d):
    fn = pl.kernel(
        _deg_body,
        out_type=jax.ShapeDtypeStruct((NC, NPAD, H), _f32),
        mesh=_sc_mesh(),
        compiler_params=pltpu.CompilerParams(use_tc_tiling_on_sc=False),
        scratch_types=[
            pltpu.VMEM_SHARED((NPAD, H), _f32),
            pltpu.VMEM((CPW, CH), jnp.int32),
            pltpu.VMEM((CH, H), _f32),
            pltpu.VMEM((RPT, H), _f32),
        ],
    )
    return fn(dst2d)


def _msg_body(g_hbm, src_hbm, dst_hbm, acc_hbm,
              g_sh, acc_sh, src_v, dst_v, msg_v, zero_v):
    cid = lax.axis_index("c")
    sid = lax.axis_index("s")
    wid = cid * NS + sid

    _zero_vmem(zero_v, RPT)
    pltpu.sync_copy(zero_v, acc_sh.at[pl.ds(sid * RPT, RPT)])
    pltpu.sync_copy(g_hbm.at[pl.ds(sid * RPT, RPT)],
                    g_sh.at[pl.ds(sid * RPT, RPT)])
    pltpu.sync_copy(src_hbm.at[wid], src_v)
    pltpu.sync_copy(dst_hbm.at[wid], dst_v)
    plsc.subcore_barrier()

    def chunk(j, carry):
        pltpu.sync_copy(g_sh.at[src_v.at[j]], msg_v)
        pltpu.sync_copy(msg_v, acc_sh.at[dst_v.at[j]], add=True)
        return carry
    lax.fori_loop(0, CPW, chunk, 0)

    plsc.subcore_barrier()
    pltpu.sync_copy(acc_sh.at[pl.ds(sid * RPT, RPT)],
                    acc_hbm.at[cid, pl.ds(sid * RPT, RPT)])


def _msg_call(g, src2d, dst2d):
    fn = pl.kernel(
        _msg_body,
        out_type=jax.ShapeDtypeStruct((NC, NPAD, H), _f32),
        mesh=_sc_mesh(),
        compiler_params=pltpu.CompilerParams(use_tc_tiling_on_sc=False),
        scratch_types=[
            pltpu.VMEM_SHARED((NPAD, H), _f32),
            pltpu.VMEM_SHARED((NPAD, H), _f32),
            pltpu.VMEM((CPW, CH), jnp.int32),
            pltpu.VMEM((CPW, CH), jnp.int32),
            pltpu.VMEM((CH, H), _f32),
            pltpu.VMEM((RPT, H), _f32),
        ],
    )
    return fn(g, src2d, dst2d)


def _k1_body(x_ref, w1_ref, deg_ref, g1_ref, dis_ref):
    dis = lax.rsqrt(deg_ref[0, :N, :] + deg_ref[1, :N, :] + 1.0)
    h = jnp.dot(x_ref[...], w1_ref[...], preferred_element_type=_f32)
    g1_ref[:N, :] = h * dis
    g1_ref[N:, :] = jnp.zeros((NPAD - N, H), _f32)
    dis_ref[...] = dis


def _k1(x, W1, deg):
    return pl.pallas_call(
        _k1_body,
        out_shape=(
            jax.ShapeDtypeStruct((NPAD, H), _f32),
            jax.ShapeDtypeStruct((N, H), _f32),
        ),
    )(x, W1, deg)


def _k2_body(acc_ref, g1_ref, dis_ref, b1_ref, w2_ref, g2_ref):
    s = acc_ref[0, :N, :] + acc_ref[1, :N, :] + g1_ref[:N, :]
    a1 = jnp.maximum(dis_ref[...] * s + b1_ref[...], 0.0)
    h2 = jnp.dot(a1, w2_ref[...], preferred_element_type=_f32)
    g2_ref[:N, :] = h2 * dis_ref[...]
    g2_ref[N:, :] = jnp.zeros((NPAD - N, H), _f32)


def _k2(acc1, g1, dis, b1, W2):
    return pl.pallas_call(
        _k2_body,
        out_shape=jax.ShapeDtypeStruct((NPAD, H), _f32),
    )(acc1, g1, dis, b1, W2)


def _k3_body(acc_ref, g2_ref, dis_ref, b2_ref, out_ref):
    s = acc_ref[0, :N, :] + acc_ref[1, :N, :] + g2_ref[:N, :]
    out_ref[...] = dis_ref[...] * s + b2_ref[...]


def _k3(acc2, g2, dis, b2):
    return pl.pallas_call(
        _k3_body,
        out_shape=jax.ShapeDtypeStruct((N, H), _f32),
    )(acc2, g2, dis, b2)


def kernel(x, edge_index, W1, b1, W2, b2):
    ei = edge_index.astype(jnp.int32)
    src, dst = ei[0], ei[1]
    npad_e = EPAD - E
    padi = jnp.arange(npad_e, dtype=jnp.int32)
    src_p = jnp.concatenate([src, padi % 16]).reshape(NW, CPW, CH)
    dst_p = jnp.concatenate([dst, N + (padi % 16)]).reshape(NW, CPW, CH)

    deg = _deg_call(dst_p)
    g1, dis = _k1(x, W1, deg)
    acc1 = _msg_call(g1, src_p, dst_p)
    g2 = _k2(acc1, g1, dis, b1.reshape(1, H), W2)
    acc2 = _msg_call(g2, src_p, dst_p)
    return _k3(acc2, g2, dis, b2.reshape(1, H))


# speedup vs baseline: 65.1318x; 1.2731x over previous
import functools

import jax
import jax.numpy as jnp
from jax import lax
from jax.experimental import pallas as pl
from jax.experimental.pallas import tpu as pltpu
from jax.experimental.pallas import tpu_sc as plsc

N = 10000
E = 320000
D = 128
H = 16

NC, NS, L = 2, 16, 16
NW = NC * NS
CH = 128
K = 8
CPW = 80
G = CPW // K
EPAD = NW * CPW * CH
NPAD = 10240
RPT = NPAD // NS

_f32 = jnp.float32


def _sc_mesh():
    return plsc.VectorSubcoreMesh(
        core_axis_name="c", subcore_axis_name="s", num_cores=NC, num_subcores=NS
    )


def _zero_vmem(zero_v, rows):
    def body(i, carry):
        zero_v[i] = jnp.zeros((L,), _f32)
        return carry
    lax.fori_loop(0, rows, body, 0)


def _deg_body(dst_hbm, deg_hbm, deg_sh, dst_v, one_v, zero_v, dsem):
    cid = lax.axis_index("c")
    sid = lax.axis_index("s")
    wid = cid * NS + sid

    def fill_zero(i, carry):
        zero_v[pl.ds(i * L, L)] = jnp.zeros((L,), _f32)
        return carry
    lax.fori_loop(0, RPT // L, fill_zero, 0)
    pltpu.sync_copy(zero_v, deg_sh.at[pl.ds(sid * RPT, RPT)])

    def fill_ones(i, carry):
        one_v[pl.ds(i * L, L)] = jnp.ones((L,), _f32)
        return carry
    lax.fori_loop(0, CH // L, fill_ones, 0)

    pltpu.sync_copy(dst_hbm.at[wid], dst_v)
    plsc.subcore_barrier()

    def group(jj, carry):
        for b in range(K):
            pltpu.async_copy(one_v, deg_sh.at[dst_v.at[jj * K + b]], dsem,
                             add=True)
        for b in range(K):
            pltpu.make_async_copy(deg_hbm.at[0, pl.ds(0, CH)], one_v,
                                  dsem).wait()
        return carry
    lax.fori_loop(0, G, group, 0)

    plsc.subcore_barrier()
    pltpu.sync_copy(deg_sh.at[pl.ds(sid * RPT, RPT)],
                    deg_hbm.at[cid, pl.ds(sid * RPT, RPT)])


def _deg_call(dst2d):
    fn = pl.kernel(
        _deg_body,
        out_type=jax.ShapeDtypeStruct((NC, NPAD), _f32),
        mesh=_sc_mesh(),
        compiler_params=pltpu.CompilerParams(use_tc_tiling_on_sc=False),
        scratch_types=[
            pltpu.VMEM_SHARED((NPAD,), _f32),
            pltpu.VMEM((CPW, CH), jnp.int32),
            pltpu.VMEM((CH,), _f32),
            pltpu.VMEM((RPT,), _f32),
            pltpu.SemaphoreType.DMA,
        ],
    )
    return fn(dst2d)


def _msg_body(g_hbm, src_hbm, dst_hbm, acc_hbm,
              g_sh, acc_sh, src_v, dst_v, msg_v, zero_v, gsem, ssem):
    cid = lax.axis_index("c")
    sid = lax.axis_index("s")
    wid = cid * NS + sid

    _zero_vmem(zero_v, RPT)
    pltpu.sync_copy(zero_v, acc_sh.at[pl.ds(sid * RPT, RPT)])
    pltpu.sync_copy(g_hbm.at[pl.ds(sid * RPT, RPT)],
                    g_sh.at[pl.ds(sid * RPT, RPT)])
    pltpu.sync_copy(src_hbm.at[wid], src_v)
    pltpu.sync_copy(dst_hbm.at[wid], dst_v)
    plsc.subcore_barrier()

    for b in range(K):
        pltpu.async_copy(g_sh.at[src_v.at[b]], msg_v.at[b], gsem)

    def group(jj, carry):
        s = (jj % 2) * K
        sn = ((jj + 1) % 2) * K

        @pl.when(jj > 0)
        def _drain_prev():
            for b in range(K):
                pltpu.make_async_copy(g_hbm.at[pl.ds(0, CH)], msg_v.at[sn + b],
                                      ssem).wait()

        @pl.when(jj < G - 1)
        def _prefetch_next():
            for b in range(K):
                jn = (jj + 1) * K + b
                pltpu.async_copy(g_sh.at[src_v.at[jn]], msg_v.at[sn + b], gsem)

        for b in range(K):
            j = jj * K + b
            pltpu.make_async_copy(g_sh.at[src_v.at[j]], msg_v.at[s + b],
                                  gsem).wait()
            pltpu.async_copy(msg_v.at[s + b], acc_sh.at[dst_v.at[j]], ssem,
                             add=True)
        return carry
    lax.fori_loop(0, G, group, 0)

    for b in range(K):
        pltpu.make_async_copy(g_hbm.at[pl.ds(0, CH)], msg_v.at[b], ssem).wait()

    plsc.subcore_barrier()
    pltpu.sync_copy(acc_sh.at[pl.ds(sid * RPT, RPT)],
                    acc_hbm.at[cid, pl.ds(sid * RPT, RPT)])


def _msg_call(g, src2d, dst2d):
    fn = pl.kernel(
        _msg_body,
        out_type=jax.ShapeDtypeStruct((NC, NPAD, H), _f32),
        mesh=_sc_mesh(),
        compiler_params=pltpu.CompilerParams(use_tc_tiling_on_sc=False),
        scratch_types=[
            pltpu.VMEM_SHARED((NPAD, H), _f32),
            pltpu.VMEM_SHARED((NPAD, H), _f32),
            pltpu.VMEM((CPW, CH), jnp.int32),
            pltpu.VMEM((CPW, CH), jnp.int32),
            pltpu.VMEM((2 * K, CH, H), _f32),
            pltpu.VMEM((RPT, H), _f32),
            pltpu.SemaphoreType.DMA,
            pltpu.SemaphoreType.DMA,
        ],
    )
    return fn(g, src2d, dst2d)


def _k1_body(x_ref, w1_ref, degt_ref, g1_ref, dis_ref):
    dis1 = lax.rsqrt(degt_ref[:N, 0:1] + degt_ref[:N, 1:2] + 1.0)
    h = jnp.dot(x_ref[...], w1_ref[...], preferred_element_type=_f32)
    g1_ref[:N, :] = h * dis1
    g1_ref[N:, :] = jnp.zeros((NPAD - N, H), _f32)
    dis_ref[...] = jnp.broadcast_to(dis1, (N, H))


def _k1(x, W1, degt):
    return pl.pallas_call(
        _k1_body,
        out_shape=(
            jax.ShapeDtypeStruct((NPAD, H), _f32),
            jax.ShapeDtypeStruct((N, H), _f32),
        ),
    )(x, W1, degt)


def _k2_body(acc_ref, g1_ref, dis_ref, b1_ref, w2_ref, g2_ref):
    s = acc_ref[0, :N, :] + acc_ref[1, :N, :] + g1_ref[:N, :]
    a1 = jnp.maximum(dis_ref[...] * s + b1_ref[...], 0.0)
    h2 = jnp.dot(a1, w2_ref[...], preferred_element_type=_f32)
    g2_ref[:N, :] = h2 * dis_ref[...]
    g2_ref[N:, :] = jnp.zeros((NPAD - N, H), _f32)


def _k2(acc1, g1, dis, b1, W2):
    return pl.pallas_call(
        _k2_body,
        out_shape=jax.ShapeDtypeStruct((NPAD, H), _f32),
    )(acc1, g1, dis, b1, W2)


def _k3_body(acc_ref, g2_ref, dis_ref, b2_ref, out_ref):
    s = acc_ref[0, :N, :] + acc_ref[1, :N, :] + g2_ref[:N, :]
    out_ref[...] = dis_ref[...] * s + b2_ref[...]


def _k3(acc2, g2, dis, b2):
    return pl.pallas_call(
        _k3_body,
        out_shape=jax.ShapeDtypeStruct((N, H), _f32),
    )(acc2, g2, dis, b2)


def kernel(x, edge_index, W1, b1, W2, b2):
    ei = edge_index.astype(jnp.int32)
    src, dst = ei[0], ei[1]
    npad_e = EPAD - E
    padi = jnp.arange(npad_e, dtype=jnp.int32)
    src_p = jnp.concatenate([src, padi % 16]).reshape(NW, CPW, CH)
    dst_p = jnp.concatenate([dst, N + (padi % 16)]).reshape(NW, CPW, CH)

    deg = _deg_call(dst_p)
    g1, dis = _k1(x, W1, deg.T)
    acc1 = _msg_call(g1, src_p, dst_p)
    g2 = _k2(acc1, g1, dis, b1.reshape(1, H), W2)
    acc2 = _msg_call(g2, src_p, dst_p)
    return _k3(acc2, g2, dis, b2.reshape(1, H))
